# baseline (speedup 1.0000x reference)
import jax
import jax.numpy as jnp
from jax import lax
from jax.experimental import pallas as pl
from jax.experimental.pallas import tpu as pltpu


_LOG2E = 1.4426950408889634


def _gat_kernel(inp_ref, w_ref, a_ref, adj_ref, out_ref,
                h_ref, fb_ref, gb_ref):
    i = pl.program_id(0)

    @pl.when(i == 0)
    def _():
        h = jnp.dot(inp_ref[...], w_ref[...], preferred_element_type=jnp.float32)
        h_ref[...] = h
        d = h.shape[1]
        f = lax.dot_general(
            h, a_ref[:, :d], (((1,), (1,)), ((), ())),
            preferred_element_type=jnp.float32)
        g = lax.dot_general(
            a_ref[:, d:], h, (((1,), (1,)), ((), ())),
            preferred_element_type=jnp.float32)
        fb_ref[:, 0:1] = f * (-_LOG2E)
        fb_ref[:, 1:2] = jnp.ones_like(f)
        gb_ref[0:1, :] = jnp.ones_like(g)
        gb_ref[1:2, :] = g * (-_LOG2E)

    blk = out_ref.shape[0]
    rows = pl.ds(i * blk, blk)
    s1 = jnp.dot(fb_ref[rows, :], gb_ref[...],
                 preferred_element_type=jnp.float32)
    e = jnp.exp2(jnp.minimum(s1, 0.01 * s1))
    e = lax.bitcast_convert_type(
        lax.bitcast_convert_type(e, jnp.int32) * adj_ref[...], jnp.float32)
    rowsum = jnp.sum(e, axis=1, keepdims=True)
    hp = jnp.dot(e, h_ref[...], preferred_element_type=jnp.float32)
    hp = hp / rowsum
    out_ref[...] = jnp.where(hp > 0.0, hp, jnp.exp(hp) - 1.0)


def kernel(input, adj, W, a):
    n, d_in = input.shape
    d_out = W.shape[1]
    blk = 512
    return pl.pallas_call(
        _gat_kernel,
        grid=(n // blk,),
        in_specs=[
            pl.BlockSpec((n, d_in), lambda i: (0, 0)),
            pl.BlockSpec((d_in, d_out), lambda i: (0, 0)),
            pl.BlockSpec((1, 2 * d_out), lambda i: (0, 0)),
            pl.BlockSpec((blk, n), lambda i: (i, 0)),
        ],
        out_specs=pl.BlockSpec((blk, d_out), lambda i: (i, 0)),
        out_shape=jax.ShapeDtypeStruct((n, d_out), jnp.float32),
        scratch_shapes=[
            pltpu.VMEM((n, d_out), jnp.float32),
            pltpu.VMEM((n, 2), jnp.float32),
            pltpu.VMEM((2, n), jnp.float32),
        ],
    )(input, W, a, adj)


# speedup vs baseline: 1.0593x; 1.0593x over previous
import jax
import jax.numpy as jnp
from jax import lax
from jax.experimental import pallas as pl
from jax.experimental.pallas import tpu as pltpu


_LOG2E = 1.4426950408889634


def _gat_kernel(inp_ref, w_ref, a_ref, adj_ref, out_ref,
                h_ref, f1_ref, g1_ref):
    i = pl.program_id(0)

    @pl.when(i == 0)
    def _():
        h = jnp.dot(inp_ref[...], w_ref[...], preferred_element_type=jnp.float32)
        h_ref[...] = h
        d = h.shape[1]
        f = lax.dot_general(
            h, a_ref[:, :d], (((1,), (1,)), ((), ())),
            preferred_element_type=jnp.float32)
        g = lax.dot_general(
            a_ref[:, d:], h, (((1,), (1,)), ((), ())),
            preferred_element_type=jnp.float32)
        f1_ref[...] = f * (-_LOG2E)
        g1_ref[...] = g * (-_LOG2E)

    blk = out_ref.shape[0]
    rows = pl.ds(i * blk, blk)
    s1 = f1_ref[rows, :] + g1_ref[...]
    e = jnp.exp2(jnp.minimum(s1, 0.01 * s1))
    e = lax.bitcast_convert_type(
        lax.bitcast_convert_type(e, jnp.int32) * adj_ref[...], jnp.float32)
    rowsum = jnp.sum(e, axis=1, keepdims=True)
    hp = jnp.dot(e, h_ref[...], preferred_element_type=jnp.float32)
    hp = hp / rowsum
    out_ref[...] = jnp.where(hp > 0.0, hp, jnp.exp(hp) - 1.0)


def kernel(input, adj, W, a):
    n, d_in = input.shape
    d_out = W.shape[1]
    blk = 512
    return pl.pallas_call(
        _gat_kernel,
        grid=(n // blk,),
        in_specs=[
            pl.BlockSpec((n, d_in), lambda i: (0, 0)),
            pl.BlockSpec((d_in, d_out), lambda i: (0, 0)),
            pl.BlockSpec((1, 2 * d_out), lambda i: (0, 0)),
            pl.BlockSpec((blk, n), lambda i: (i, 0)),
        ],
        out_specs=pl.BlockSpec((blk, d_out), lambda i: (i, 0)),
        out_shape=jax.ShapeDtypeStruct((n, d_out), jnp.float32),
        scratch_shapes=[
            pltpu.VMEM((n, d_out), jnp.float32),
            pltpu.VMEM((n, 1), jnp.float32),
            pltpu.VMEM((1, n), jnp.float32),
        ],
    )(input, W, a, adj)


# speedup vs baseline: 1.1030x; 1.0412x over previous
import jax
import jax.numpy as jnp
from jax import lax
from jax.experimental import pallas as pl
from jax.experimental.pallas import tpu as pltpu


_LOG2E = 1.4426950408889634


def _gat_kernel(inp_ref, w_ref, a_ref, adj_ref, out_ref,
                h_ref, f1_ref, g1_ref):
    i = pl.program_id(0)

    @pl.when(i == 0)
    def _():
        h = jnp.dot(inp_ref[...], w_ref[...], preferred_element_type=jnp.float32)
        h_ref[...] = h
        d = h.shape[1]
        f = lax.dot_general(
            h, a_ref[:, :d], (((1,), (1,)), ((), ())),
            preferred_element_type=jnp.float32)
        g = lax.dot_general(
            a_ref[:, d:], h, (((1,), (1,)), ((), ())),
            preferred_element_type=jnp.float32)
        f1_ref[...] = f * (-_LOG2E)
        g1_ref[...] = g * (-_LOG2E)

    blk = out_ref.shape[0]
    n = h_ref.shape[0]
    d = h_ref.shape[1]
    rows = pl.ds(i * blk, blk)
    f1 = f1_ref[rows, :]
    cc = 256
    hp = jnp.zeros((blk, d), jnp.float32)
    rowsum = jnp.zeros((blk, 1), jnp.float32)
    for k in range(n // cc):
        cols = pl.ds(k * cc, cc)
        s1 = f1 + g1_ref[:, cols]
        e = jnp.exp2(jnp.minimum(s1, 0.01 * s1))
        e = lax.bitcast_convert_type(
            lax.bitcast_convert_type(e, jnp.int32) * adj_ref[:, cols],
            jnp.float32)
        rowsum = rowsum + jnp.sum(e, axis=1, keepdims=True)
        hp = hp + jnp.dot(e, h_ref[cols, :], preferred_element_type=jnp.float32)
    hp = hp / rowsum
    out_ref[...] = jnp.where(hp > 0.0, hp, jnp.exp(hp) - 1.0)


def kernel(input, adj, W, a):
    n, d_in = input.shape
    d_out = W.shape[1]
    blk = 512
    return pl.pallas_call(
        _gat_kernel,
        grid=(n // blk,),
        in_specs=[
            pl.BlockSpec((n, d_in), lambda i: (0, 0)),
            pl.BlockSpec((d_in, d_out), lambda i: (0, 0)),
            pl.BlockSpec((1, 2 * d_out), lambda i: (0, 0)),
            pl.BlockSpec((blk, n), lambda i: (i, 0)),
        ],
        out_specs=pl.BlockSpec((blk, d_out), lambda i: (i, 0)),
        out_shape=jax.ShapeDtypeStruct((n, d_out), jnp.float32),
        scratch_shapes=[
            pltpu.VMEM((n, d_out), jnp.float32),
            pltpu.VMEM((n, 1), jnp.float32),
            pltpu.VMEM((1, n), jnp.float32),
        ],
    )(input, W, a, adj)
